# baseline (speedup 1.0000x reference)
import functools

import jax
import jax.numpy as jnp
from jax import lax
from jax.experimental import pallas as pl
from jax.experimental.pallas import tpu as pltpu
from jax.experimental.pallas import tpu_sc as plsc

_NBINS = 50
_K1 = 0.1
_NB = 16
_R0 = 4096 // _NB
_R1 = 2048 // _NB
_COLS = 4096

_NW = 32
_CROWS = 4
_NBUF = 4
_NBPAD = 64
_NREG = 16
_REGW = _NBPAD * 16


def _tc_reduce_kernel(p0, g0, p1, g1, scal, splat, acc):
    i = pl.program_id(0)

    @pl.when(i == 0)
    def _init():
        acc[0] = jnp.inf
        acc[1] = -jnp.inf
        acc[2] = 0.0
        acc[3] = jnp.inf
        acc[4] = -jnp.inf
        acc[5] = 0.0

    x0 = p0[...]
    acc[0] = jnp.minimum(acc[0], jnp.min(x0))
    acc[1] = jnp.maximum(acc[1], jnp.max(x0))
    gg0 = g0[...]
    acc[2] = acc[2] + jnp.sum(gg0 * gg0)
    x1 = p1[...]
    acc[3] = jnp.minimum(acc[3], jnp.min(x1))
    acc[4] = jnp.maximum(acc[4], jnp.max(x1))
    gg1 = g1[...]
    acc[5] = acc[5] + jnp.sum(gg1 * gg1)

    @pl.when(i == _NB - 1)
    def _fin():
        for k in range(6):
            scal[k] = acc[k]
        s0 = _NBINS / (acc[1] - acc[0] + 1e-12)
        s1 = _NBINS / (acc[4] - acc[3] + 1e-12)
        z = jnp.zeros((8, 128), jnp.float32)
        splat[...] = z
        o = jnp.zeros((1, 128), jnp.float32)
        splat[0:1, :] = o + acc[0]
        splat[1:2, :] = o + s0
        splat[2:3, :] = o + acc[3]
        splat[3:4, :] = o + s1


def _tc_reduce(param0, grad0, param1, grad1):
    return pl.pallas_call(
        _tc_reduce_kernel,
        grid=(_NB,),
        in_specs=[
            pl.BlockSpec((_R0, _COLS), lambda i: (i, 0)),
            pl.BlockSpec((_R0, _COLS), lambda i: (i, 0)),
            pl.BlockSpec((_R1, _COLS), lambda i: (i, 0)),
            pl.BlockSpec((_R1, _COLS), lambda i: (i, 0)),
        ],
        out_specs=[
            pl.BlockSpec((8,), lambda i: (0,), memory_space=pltpu.SMEM),
            pl.BlockSpec((8, 128), lambda i: (0, 0)),
        ],
        out_shape=[
            jax.ShapeDtypeStruct((8,), jnp.float32),
            jax.ShapeDtypeStruct((8, 128), jnp.float32),
        ],
        scratch_shapes=[pltpu.SMEM((8,), jnp.float32)],
    )(param0, grad0, param1, grad1)


def _sc_hist_body(p0_hbm, p1_hbm, splat_hbm, out_hbm,
                  b0, b1, b2, b3, consts_v, outbuf, hist,
                  s0_, s1_, s2_, s3_):
    bufs = (b0, b1, b2, b3)
    sems = (s0_, s1_, s2_, s3_)
    wid = lax.axis_index("s") * 2 + lax.axis_index("c")

    pltpu.sync_copy(splat_hbm, consts_v)
    mn0 = consts_v[0, pl.ds(0, 16)]
    s0 = consts_v[1, pl.ds(0, 16)]
    mn1 = consts_v[2, pl.ds(0, 16)]
    s1 = consts_v[3, pl.ds(0, 16)]

    zeros16 = jnp.zeros((16,), jnp.float32)
    ones16 = zeros16 + 1.0
    lane = lax.broadcasted_iota(jnp.int32, (16,), 0)

    @plsc.parallel_loop(0, 2 * _NREG * _REGW // 16)
    def _zero(k):
        hist[pl.ds(k * 16, 16)] = zeros16

    def process_chunk(buf, mn, s, toff):
        @plsc.parallel_loop(0, _CROWS * _COLS // 16, unroll=8)
        def _body(i):
            row = i // (_COLS // 16)
            col = (i % (_COLS // 16)) * 16
            roff = (i % _NREG) * _REGW + toff
            x = buf[row, pl.ds(col, 16)]
            t = (x - mn) * s
            q = t.astype(jnp.int32)
            plsc.addupdate_scatter(
                hist.at[pl.ds(roff, _REGW)],
                [(q << 4) + lane], ones16)

    def run_tensor(hbm, rows_per_tile, mn, s, toff):
        base = wid * rows_per_tile
        nsteps = rows_per_tile // _CROWS

        def copy_step(st, b):
            return pltpu.make_async_copy(
                hbm.at[pl.ds(base + st * _CROWS, _CROWS), :],
                bufs[b], sems[b])

        for b in range(_NBUF - 1):
            copy_step(b, b).start()

        def gbody(g, carry):
            st = _NBUF * g
            for b in range(_NBUF):
                nxt = st + b + _NBUF - 1
                @pl.when(nxt < nsteps)
                def _():
                    copy_step(nxt, (b + _NBUF - 1) % _NBUF).start()
                copy_step(st + b, b).wait()
                process_chunk(bufs[b], mn, s, toff)
            return carry

        lax.fori_loop(0, nsteps // _NBUF, gbody, 0)

    run_tensor(p0_hbm, 4096 // _NW, mn0, s0, 0)
    run_tensor(p1_hbm, 2048 // _NW, mn1, s1, _NREG * _REGW)

    lane128 = lane * 128

    def bbody(b, carry):
        for t in range(2):
            accs = [zeros16, zeros16, zeros16, zeros16]
            for r in range(_NREG):
                off = t * _NREG * _REGW + r * _REGW + b * 16
                accs[r % 4] = accs[r % 4] + hist[pl.ds(off, 16)]
            acc = (accs[0] + accs[1]) + (accs[2] + accs[3])
            plsc.store_scatter(outbuf, [lane128 + (t * 64 + b)], acc)
        return carry
    lax.fori_loop(0, _NBPAD, bbody, 0)

    pltpu.sync_copy(outbuf, out_hbm.at[wid])


def _sc_hist(param0, param1, splat):
    mesh = plsc.VectorSubcoreMesh(core_axis_name="c", subcore_axis_name="s")
    f = functools.partial(
        pl.kernel,
        mesh=mesh,
        out_type=jax.ShapeDtypeStruct((_NW, 2 * _REGW), jnp.float32),
        compiler_params=pltpu.CompilerParams(needs_layout_passes=False),
        scratch_types=[
            pltpu.VMEM((_CROWS, _COLS), jnp.float32),
            pltpu.VMEM((_CROWS, _COLS), jnp.float32),
            pltpu.VMEM((_CROWS, _COLS), jnp.float32),
            pltpu.VMEM((_CROWS, _COLS), jnp.float32),
            pltpu.VMEM((8, 128), jnp.float32),
            pltpu.VMEM((2 * _REGW,), jnp.float32),
            pltpu.VMEM((2 * _NREG * _REGW,), jnp.float32),
            pltpu.SemaphoreType.DMA,
            pltpu.SemaphoreType.DMA,
            pltpu.SemaphoreType.DMA,
            pltpu.SemaphoreType.DMA,
        ],
    )(_sc_hist_body)
    return f(param0, param1, splat)


def _tc_final_kernel(scal, parts, out):
    h = jnp.sum(parts[...].reshape(16 * _NW, 128), axis=0, keepdims=True)
    lanes = lax.broadcasted_iota(jnp.int32, (1, 128), 1)
    c50_0 = jnp.sum(jnp.where(lanes == 50, h, 0.0))
    c50_1 = jnp.sum(jnp.where(lanes == 114, h, 0.0))
    h = h + jnp.where(lanes == 49, c50_0, 0.0)
    h = h + jnp.where(lanes == 113, c50_1, 0.0)
    h = jnp.where((lanes % 64) == 50, 0.0, h)

    def entropy(hh):
        tot = jnp.sum(hh)
        p = hh / (tot + 1e-10)
        return -jnp.sum(p * jnp.log(p + 1e-10))

    e0 = entropy(h[:, 0:64])
    e1 = entropy(h[:, 64:128])
    rho0 = scal[2] / (1.0 + e0)
    rho1 = scal[5] / (1.0 + e1)
    rho = 0.5 * (rho0 + rho1)
    out[0] = -_K1 * jnp.log(rho + 1e-10)
    out[1] = rho
    out[2] = 1.0 - _K1 * rho


def _tc_final(scal, parts):
    return pl.pallas_call(
        _tc_final_kernel,
        in_specs=[
            pl.BlockSpec(memory_space=pltpu.SMEM),
            pl.BlockSpec(memory_space=pltpu.VMEM),
        ],
        out_specs=pl.BlockSpec(memory_space=pltpu.SMEM),
        out_shape=jax.ShapeDtypeStruct((4,), jnp.float32),
    )(scal, parts)


def kernel(param0, grad0, param1, grad1):
    scal, splat = _tc_reduce(param0, grad0, param1, grad1)
    parts = _sc_hist(param0, param1, splat)
    out = _tc_final(scal, parts)
    return (out[0], out[1], out[2])


# speedup vs baseline: 1.1367x; 1.1367x over previous
import functools

import jax
import jax.numpy as jnp
from jax import lax
from jax.experimental import pallas as pl
from jax.experimental.pallas import tpu as pltpu
from jax.experimental.pallas import tpu_sc as plsc

_NBINS = 50
_K1 = 0.1
_NB = 16
_R0 = 4096 // _NB
_R1 = 2048 // _NB
_COLS = 4096

_NW = 32
_CROWS = 4
_NBUF = 4
_NBPAD = 64
_NREG = 16
_REGW = _NBPAD * 16


def _tc_minmax_kernel(p0, p1, splat, acc):
    i = pl.program_id(0)

    @pl.when(i == 0)
    def _init():
        acc[0] = jnp.inf
        acc[1] = -jnp.inf
        acc[2] = jnp.inf
        acc[3] = -jnp.inf

    x0 = p0[...]
    acc[0] = jnp.minimum(acc[0], jnp.min(x0))
    acc[1] = jnp.maximum(acc[1], jnp.max(x0))
    x1 = p1[...]
    acc[2] = jnp.minimum(acc[2], jnp.min(x1))
    acc[3] = jnp.maximum(acc[3], jnp.max(x1))

    @pl.when(i == _NB - 1)
    def _fin():
        s0 = _NBINS / (acc[1] - acc[0] + 1e-12)
        s1 = _NBINS / (acc[3] - acc[2] + 1e-12)
        z = jnp.zeros((8, 128), jnp.float32)
        splat[...] = z
        o = jnp.zeros((1, 128), jnp.float32)
        splat[0:1, :] = o + acc[0]
        splat[1:2, :] = o + s0
        splat[2:3, :] = o + acc[2]
        splat[3:4, :] = o + s1


def _tc_minmax(param0, param1):
    return pl.pallas_call(
        _tc_minmax_kernel,
        grid=(_NB,),
        in_specs=[
            pl.BlockSpec((_R0, _COLS), lambda i: (i, 0)),
            pl.BlockSpec((_R1, _COLS), lambda i: (i, 0)),
        ],
        out_specs=pl.BlockSpec((8, 128), lambda i: (0, 0)),
        out_shape=jax.ShapeDtypeStruct((8, 128), jnp.float32),
        scratch_shapes=[pltpu.SMEM((8,), jnp.float32)],
    )(param0, param1)


def _tc_sumsq_kernel(g0, g1, out, acc):
    i = pl.program_id(0)

    @pl.when(i == 0)
    def _init():
        acc[0] = 0.0
        acc[1] = 0.0

    gg0 = g0[...]
    acc[0] = acc[0] + jnp.sum(gg0 * gg0)
    gg1 = g1[...]
    acc[1] = acc[1] + jnp.sum(gg1 * gg1)

    @pl.when(i == _NB - 1)
    def _fin():
        out[0] = acc[0]
        out[1] = acc[1]


def _tc_sumsq(grad0, grad1):
    return pl.pallas_call(
        _tc_sumsq_kernel,
        grid=(_NB,),
        in_specs=[
            pl.BlockSpec((_R0, _COLS), lambda i: (i, 0)),
            pl.BlockSpec((_R1, _COLS), lambda i: (i, 0)),
        ],
        out_specs=pl.BlockSpec((2,), lambda i: (0,), memory_space=pltpu.SMEM),
        out_shape=jax.ShapeDtypeStruct((2,), jnp.float32),
        scratch_shapes=[pltpu.SMEM((2,), jnp.float32)],
    )(grad0, grad1)


def _sc_hist_body(p0_hbm, p1_hbm, splat_hbm, out_hbm,
                  b0, b1, b2, b3, consts_v, outbuf, hist,
                  s0_, s1_, s2_, s3_):
    bufs = (b0, b1, b2, b3)
    sems = (s0_, s1_, s2_, s3_)
    wid = lax.axis_index("s") * 2 + lax.axis_index("c")

    pltpu.sync_copy(splat_hbm, consts_v)
    mn0 = consts_v[0, pl.ds(0, 16)]
    s0 = consts_v[1, pl.ds(0, 16)]
    mn1 = consts_v[2, pl.ds(0, 16)]
    s1 = consts_v[3, pl.ds(0, 16)]

    zeros16 = jnp.zeros((16,), jnp.float32)
    ones16 = zeros16 + 1.0
    lane = lax.broadcasted_iota(jnp.int32, (16,), 0)

    @plsc.parallel_loop(0, 2 * _NREG * _REGW // 16)
    def _zero(k):
        hist[pl.ds(k * 16, 16)] = zeros16

    def process_chunk(buf, mn, s, toff):
        @plsc.parallel_loop(0, _CROWS * _COLS // 16, unroll=8)
        def _body(i):
            row = i // (_COLS // 16)
            col = (i % (_COLS // 16)) * 16
            roff = (i % _NREG) * _REGW + toff
            x = buf[row, pl.ds(col, 16)]
            t = (x - mn) * s
            q = t.astype(jnp.int32)
            plsc.addupdate_scatter(
                hist.at[pl.ds(roff, _REGW)],
                [(q << 4) + lane], ones16)

    def run_tensor(hbm, rows_per_tile, mn, s, toff):
        base = wid * rows_per_tile
        nsteps = rows_per_tile // _CROWS

        def copy_step(st, b):
            return pltpu.make_async_copy(
                hbm.at[pl.ds(base + st * _CROWS, _CROWS), :],
                bufs[b], sems[b])

        for b in range(_NBUF - 1):
            copy_step(b, b).start()

        def gbody(g, carry):
            st = _NBUF * g
            for b in range(_NBUF):
                nxt = st + b + _NBUF - 1
                @pl.when(nxt < nsteps)
                def _():
                    copy_step(nxt, (b + _NBUF - 1) % _NBUF).start()
                copy_step(st + b, b).wait()
                process_chunk(bufs[b], mn, s, toff)
            return carry

        lax.fori_loop(0, nsteps // _NBUF, gbody, 0)

    run_tensor(p0_hbm, 4096 // _NW, mn0, s0, 0)
    run_tensor(p1_hbm, 2048 // _NW, mn1, s1, _NREG * _REGW)

    lane128 = lane * 128

    def bbody(b, carry):
        for t in range(2):
            accs = [zeros16, zeros16, zeros16, zeros16]
            for r in range(_NREG):
                off = t * _NREG * _REGW + r * _REGW + b * 16
                accs[r % 4] = accs[r % 4] + hist[pl.ds(off, 16)]
            acc = (accs[0] + accs[1]) + (accs[2] + accs[3])
            plsc.store_scatter(outbuf, [lane128 + (t * 64 + b)], acc)
        return carry
    lax.fori_loop(0, _NBPAD, bbody, 0)

    pltpu.sync_copy(outbuf, out_hbm.at[wid])


def _sc_hist(param0, param1, splat):
    mesh = plsc.VectorSubcoreMesh(core_axis_name="c", subcore_axis_name="s")
    f = functools.partial(
        pl.kernel,
        mesh=mesh,
        out_type=jax.ShapeDtypeStruct((_NW, 2 * _REGW), jnp.float32),
        compiler_params=pltpu.CompilerParams(needs_layout_passes=False),
        scratch_types=[
            pltpu.VMEM((_CROWS, _COLS), jnp.float32),
            pltpu.VMEM((_CROWS, _COLS), jnp.float32),
            pltpu.VMEM((_CROWS, _COLS), jnp.float32),
            pltpu.VMEM((_CROWS, _COLS), jnp.float32),
            pltpu.VMEM((8, 128), jnp.float32),
            pltpu.VMEM((2 * _REGW,), jnp.float32),
            pltpu.VMEM((2 * _NREG * _REGW,), jnp.float32),
            pltpu.SemaphoreType.DMA,
            pltpu.SemaphoreType.DMA,
            pltpu.SemaphoreType.DMA,
            pltpu.SemaphoreType.DMA,
        ],
    )(_sc_hist_body)
    return f(param0, param1, splat)


def _tc_final_kernel(scal, parts, out):
    h = jnp.sum(parts[...].reshape(16 * _NW, 128), axis=0, keepdims=True)
    lanes = lax.broadcasted_iota(jnp.int32, (1, 128), 1)
    c50_0 = jnp.sum(jnp.where(lanes == 50, h, 0.0))
    c50_1 = jnp.sum(jnp.where(lanes == 114, h, 0.0))
    h = h + jnp.where(lanes == 49, c50_0, 0.0)
    h = h + jnp.where(lanes == 113, c50_1, 0.0)
    h = jnp.where((lanes % 64) == 50, 0.0, h)

    def entropy(hh):
        tot = jnp.sum(hh)
        p = hh / (tot + 1e-10)
        return -jnp.sum(p * jnp.log(p + 1e-10))

    e0 = entropy(h[:, 0:64])
    e1 = entropy(h[:, 64:128])
    rho0 = scal[0] / (1.0 + e0)
    rho1 = scal[1] / (1.0 + e1)
    rho = 0.5 * (rho0 + rho1)
    out[0] = -_K1 * jnp.log(rho + 1e-10)
    out[1] = rho
    out[2] = 1.0 - _K1 * rho


def _tc_final(scal, parts):
    return pl.pallas_call(
        _tc_final_kernel,
        in_specs=[
            pl.BlockSpec(memory_space=pltpu.SMEM),
            pl.BlockSpec(memory_space=pltpu.VMEM),
        ],
        out_specs=pl.BlockSpec(memory_space=pltpu.SMEM),
        out_shape=jax.ShapeDtypeStruct((4,), jnp.float32),
    )(scal, parts)


def kernel(param0, grad0, param1, grad1):
    splat = _tc_minmax(param0, param1)
    parts = _sc_hist(param0, param1, splat)
    ss = _tc_sumsq(grad0, grad1)
    out = _tc_final(ss, parts)
    return (out[0], out[1], out[2])


# speedup vs baseline: 1.2257x; 1.0783x over previous
import functools

import jax
import jax.numpy as jnp
from jax import lax
from jax.experimental import pallas as pl
from jax.experimental.pallas import tpu as pltpu
from jax.experimental.pallas import tpu_sc as plsc

_NBINS = 50
_K1 = 0.1
_NB = 16
_R0 = 4096 // _NB
_R1 = 2048 // _NB
_COLS = 4096

_NW = 32
_CROWS = 4
_NBUF = 4
_NBPAD = 64
_NREG = 1
_REGW = _NBPAD * 16


def _tc_minmax_kernel(p0, p1, splat, acc):
    i = pl.program_id(0)

    @pl.when(i == 0)
    def _init():
        acc[0] = jnp.inf
        acc[1] = -jnp.inf
        acc[2] = jnp.inf
        acc[3] = -jnp.inf

    x0 = p0[...]
    acc[0] = jnp.minimum(acc[0], jnp.min(x0))
    acc[1] = jnp.maximum(acc[1], jnp.max(x0))
    x1 = p1[...]
    acc[2] = jnp.minimum(acc[2], jnp.min(x1))
    acc[3] = jnp.maximum(acc[3], jnp.max(x1))

    @pl.when(i == _NB - 1)
    def _fin():
        s0 = _NBINS / (acc[1] - acc[0] + 1e-12)
        s1 = _NBINS / (acc[3] - acc[2] + 1e-12)
        z = jnp.zeros((8, 128), jnp.float32)
        splat[...] = z
        o = jnp.zeros((1, 128), jnp.float32)
        splat[0:1, :] = o + acc[0]
        splat[1:2, :] = o + s0
        splat[2:3, :] = o + acc[2]
        splat[3:4, :] = o + s1


def _tc_minmax(param0, param1):
    return pl.pallas_call(
        _tc_minmax_kernel,
        grid=(_NB,),
        in_specs=[
            pl.BlockSpec((_R0, _COLS), lambda i: (i, 0)),
            pl.BlockSpec((_R1, _COLS), lambda i: (i, 0)),
        ],
        out_specs=pl.BlockSpec((8, 128), lambda i: (0, 0)),
        out_shape=jax.ShapeDtypeStruct((8, 128), jnp.float32),
        scratch_shapes=[pltpu.SMEM((8,), jnp.float32)],
    )(param0, param1)


def _tc_sumsq_kernel(g0, g1, out, acc):
    i = pl.program_id(0)

    @pl.when(i == 0)
    def _init():
        acc[0] = 0.0
        acc[1] = 0.0

    gg0 = g0[...]
    acc[0] = acc[0] + jnp.sum(gg0 * gg0)
    gg1 = g1[...]
    acc[1] = acc[1] + jnp.sum(gg1 * gg1)

    @pl.when(i == _NB - 1)
    def _fin():
        out[0] = acc[0]
        out[1] = acc[1]


def _tc_sumsq(grad0, grad1):
    return pl.pallas_call(
        _tc_sumsq_kernel,
        grid=(_NB,),
        in_specs=[
            pl.BlockSpec((_R0, _COLS), lambda i: (i, 0)),
            pl.BlockSpec((_R1, _COLS), lambda i: (i, 0)),
        ],
        out_specs=pl.BlockSpec((2,), lambda i: (0,), memory_space=pltpu.SMEM),
        out_shape=jax.ShapeDtypeStruct((2,), jnp.float32),
        scratch_shapes=[pltpu.SMEM((2,), jnp.float32)],
    )(grad0, grad1)


def _sc_hist_body(p0_hbm, p1_hbm, splat_hbm, out_hbm,
                  b0, b1, b2, b3, consts_v, outbuf, hist,
                  s0_, s1_, s2_, s3_):
    bufs = (b0, b1, b2, b3)
    sems = (s0_, s1_, s2_, s3_)
    wid = lax.axis_index("s") * 2 + lax.axis_index("c")

    pltpu.sync_copy(splat_hbm, consts_v)
    mn0 = consts_v[0, pl.ds(0, 16)]
    s0 = consts_v[1, pl.ds(0, 16)]
    mn1 = consts_v[2, pl.ds(0, 16)]
    s1 = consts_v[3, pl.ds(0, 16)]

    zeros16 = jnp.zeros((16,), jnp.float32)
    ones16 = zeros16 + 1.0
    lane = lax.broadcasted_iota(jnp.int32, (16,), 0)

    @plsc.parallel_loop(0, 2 * _NREG * _REGW // 16)
    def _zero(k):
        hist[pl.ds(k * 16, 16)] = zeros16

    def process_chunk(buf, mn, s, toff):
        @plsc.parallel_loop(0, _CROWS * _COLS // 16, unroll=8)
        def _body(i):
            row = i // (_COLS // 16)
            col = (i % (_COLS // 16)) * 16
            roff = (i % _NREG) * _REGW + toff
            x = buf[row, pl.ds(col, 16)]
            t = (x - mn) * s
            q = t.astype(jnp.int32)
            plsc.addupdate_scatter(
                hist.at[pl.ds(roff, _REGW)],
                [(q << 4) + lane], ones16)

    def run_tensor(hbm, rows_per_tile, mn, s, toff):
        base = wid * rows_per_tile
        nsteps = rows_per_tile // _CROWS

        def copy_step(st, b):
            return pltpu.make_async_copy(
                hbm.at[pl.ds(base + st * _CROWS, _CROWS), :],
                bufs[b], sems[b])

        for b in range(_NBUF - 1):
            copy_step(b, b).start()

        def gbody(g, carry):
            st = _NBUF * g
            for b in range(_NBUF):
                nxt = st + b + _NBUF - 1
                @pl.when(nxt < nsteps)
                def _():
                    copy_step(nxt, (b + _NBUF - 1) % _NBUF).start()
                copy_step(st + b, b).wait()
                process_chunk(bufs[b], mn, s, toff)
            return carry

        lax.fori_loop(0, nsteps // _NBUF, gbody, 0)

    run_tensor(p0_hbm, 4096 // _NW, mn0, s0, 0)
    run_tensor(p1_hbm, 2048 // _NW, mn1, s1, _NREG * _REGW)

    lane128 = lane * 128

    def bbody(b, carry):
        for t in range(2):
            accs = [zeros16, zeros16, zeros16, zeros16]
            for r in range(_NREG):
                off = t * _NREG * _REGW + r * _REGW + b * 16
                accs[r % 4] = accs[r % 4] + hist[pl.ds(off, 16)]
            acc = (accs[0] + accs[1]) + (accs[2] + accs[3])
            plsc.store_scatter(outbuf, [lane128 + (t * 64 + b)], acc)
        return carry
    lax.fori_loop(0, _NBPAD, bbody, 0)

    pltpu.sync_copy(outbuf, out_hbm.at[wid])


def _sc_hist(param0, param1, splat):
    mesh = plsc.VectorSubcoreMesh(core_axis_name="c", subcore_axis_name="s")
    f = functools.partial(
        pl.kernel,
        mesh=mesh,
        out_type=jax.ShapeDtypeStruct((_NW, 2 * _REGW), jnp.float32),
        compiler_params=pltpu.CompilerParams(needs_layout_passes=False),
        scratch_types=[
            pltpu.VMEM((_CROWS, _COLS), jnp.float32),
            pltpu.VMEM((_CROWS, _COLS), jnp.float32),
            pltpu.VMEM((_CROWS, _COLS), jnp.float32),
            pltpu.VMEM((_CROWS, _COLS), jnp.float32),
            pltpu.VMEM((8, 128), jnp.float32),
            pltpu.VMEM((2 * _REGW,), jnp.float32),
            pltpu.VMEM((2 * _NREG * _REGW,), jnp.float32),
            pltpu.SemaphoreType.DMA,
            pltpu.SemaphoreType.DMA,
            pltpu.SemaphoreType.DMA,
            pltpu.SemaphoreType.DMA,
        ],
    )(_sc_hist_body)
    return f(param0, param1, splat)


def _tc_final_kernel(scal, parts, out):
    h = jnp.sum(parts[...].reshape(16 * _NW, 128), axis=0, keepdims=True)
    lanes = lax.broadcasted_iota(jnp.int32, (1, 128), 1)
    c50_0 = jnp.sum(jnp.where(lanes == 50, h, 0.0))
    c50_1 = jnp.sum(jnp.where(lanes == 114, h, 0.0))
    h = h + jnp.where(lanes == 49, c50_0, 0.0)
    h = h + jnp.where(lanes == 113, c50_1, 0.0)
    h = jnp.where((lanes % 64) == 50, 0.0, h)

    def entropy(hh):
        tot = jnp.sum(hh)
        p = hh / (tot + 1e-10)
        return -jnp.sum(p * jnp.log(p + 1e-10))

    e0 = entropy(h[:, 0:64])
    e1 = entropy(h[:, 64:128])
    rho0 = scal[0] / (1.0 + e0)
    rho1 = scal[1] / (1.0 + e1)
    rho = 0.5 * (rho0 + rho1)
    out[0] = -_K1 * jnp.log(rho + 1e-10)
    out[1] = rho
    out[2] = 1.0 - _K1 * rho


def _tc_final(scal, parts):
    return pl.pallas_call(
        _tc_final_kernel,
        in_specs=[
            pl.BlockSpec(memory_space=pltpu.SMEM),
            pl.BlockSpec(memory_space=pltpu.VMEM),
        ],
        out_specs=pl.BlockSpec(memory_space=pltpu.SMEM),
        out_shape=jax.ShapeDtypeStruct((4,), jnp.float32),
    )(scal, parts)


def kernel(param0, grad0, param1, grad1):
    splat = _tc_minmax(param0, param1)
    parts = _sc_hist(param0, param1, splat)
    ss = _tc_sumsq(grad0, grad1)
    out = _tc_final(ss, parts)
    return (out[0], out[1], out[2])
